# Initial kernel scaffold; baseline (speedup 1.0000x reference)
import functools

import jax
import jax.numpy as jnp
from jax import lax
from jax.experimental import pallas as pl
from jax.experimental.pallas import tpu as pltpu
from jax.experimental.pallas import tpu_sc as plsc

N = 10000
E = 320000
D = 128

L = 16
NC = 2
NS = 16

CHUNK = 80
EDGES_PER_TILE = E // (NC * NS)
NCHUNK = EDGES_PER_TILE // CHUNK
ROWS_PER_TILE = N // NS
ZROWS = 125


def _sc_spmm_body(dst_hbm, src_hbm, val_hbm, x_hbm, out_hbm,
                  src_v, dst_v, val_v, rows_v, zbuf, acc_sh, sem):
    c = lax.axis_index("c")
    s = lax.axis_index("s")

    zv = jnp.zeros((L,), jnp.float32)

    def zrow(r, carry):
        for j in range(D // L):
            zbuf[r, pl.ds(j * L, L)] = zv
        return carry

    lax.fori_loop(0, ZROWS, zrow, 0)
    for k in range(ROWS_PER_TILE // ZROWS):
        pltpu.sync_copy(zbuf,
                        acc_sh.at[pl.ds(s * ROWS_PER_TILE + k * ZROWS, ZROWS)])
    plsc.subcore_barrier()

    base = (c * NS + s) * EDGES_PER_TILE

    def chunk_body(i, carry):
        e0 = base + i * CHUNK
        pltpu.sync_copy(src_hbm.at[pl.ds(e0, CHUNK)], src_v)
        pltpu.sync_copy(dst_hbm.at[pl.ds(e0, CHUNK)], dst_v)
        pltpu.sync_copy(val_hbm.at[pl.ds(e0, CHUNK)], val_v)
        pltpu.async_copy(x_hbm.at[src_v], rows_v, sem).wait()

        def scale(e, cc):
            lanes = jnp.full((L,), e, dtype=jnp.int32)
            vv = plsc.load_gather(val_v, [lanes])
            for j in range(D // L):
                sl = pl.ds(j * L, L)
                rows_v[e, sl] = rows_v[e, sl] * vv
            return cc

        lax.fori_loop(0, CHUNK, scale, 0)
        pltpu.sync_copy(rows_v, acc_sh.at[dst_v], add=True)
        return carry

    lax.fori_loop(0, NCHUNK, chunk_body, 0)
    plsc.subcore_barrier()

    r0 = s * ROWS_PER_TILE
    pltpu.sync_copy(acc_sh.at[pl.ds(r0, ROWS_PER_TILE)],
                    out_hbm.at[c, pl.ds(r0, ROWS_PER_TILE)])


_sc_spmm = functools.partial(
    pl.kernel,
    out_type=jax.ShapeDtypeStruct((NC, N, D), jnp.float32),
    mesh=plsc.VectorSubcoreMesh(core_axis_name="c", subcore_axis_name="s"),
    scratch_types=[
        pltpu.VMEM((CHUNK,), jnp.int32),
        pltpu.VMEM((CHUNK,), jnp.int32),
        pltpu.VMEM((CHUNK,), jnp.float32),
        pltpu.VMEM((CHUNK, D), jnp.float32),
        pltpu.VMEM((ZROWS, D), jnp.float32),
        pltpu.VMEM_SHARED((N, D), jnp.float32),
        pltpu.SemaphoreType.DMA,
    ],
)(_sc_spmm_body)


BLK = 1000


def _linear_body(p_ref, w_ref, b_ref, o_ref):
    acc = p_ref[0] + p_ref[1]
    o_ref[...] = lax.dot_general(
        acc, w_ref[...], (((1,), (1,)), ((), ())),
        preferred_element_type=jnp.float32) + b_ref[...]


_linear = pl.pallas_call(
    _linear_body,
    grid=(N // BLK,),
    in_specs=[
        pl.BlockSpec((NC, BLK, D), lambda i: (0, i, 0)),
        pl.BlockSpec((D, D), lambda i: (0, 0)),
        pl.BlockSpec((1, D), lambda i: (0, 0)),
    ],
    out_specs=pl.BlockSpec((BLK, D), lambda i: (i, 0)),
    out_shape=jax.ShapeDtypeStruct((N, D), jnp.float32),
)


def kernel(A_indices, A_values, X, W, b):
    dst = A_indices[0].astype(jnp.int32)
    src = A_indices[1].astype(jnp.int32)
    partials = _sc_spmm(dst, src, A_values, X)
    return _linear(partials, W, b.reshape(1, D))


# speedup vs baseline: 4.5460x; 4.5460x over previous
import functools

import jax
import jax.numpy as jnp
from jax import lax
from jax.experimental import pallas as pl
from jax.experimental.pallas import tpu as pltpu
from jax.experimental.pallas import tpu_sc as plsc

N = 10000
E = 320000
D = 128

L = 16
NC = 2
NS = 16

CHUNK = 80
EDGES_PER_TILE = E // (NC * NS)
NCHUNK = EDGES_PER_TILE // CHUNK
ROWS_PER_TILE = N // NS
ZROWS = 125
WROWS = 624


def _sc_spmm_body(dst_hbm, src_hbm, val_hbm, x_hbm, out_hbm,
                  src_v, dst_v, val_v, rows_v, zbuf, acc_sh, sem):
    c = lax.axis_index("c")
    s = lax.axis_index("s")

    zv = jnp.zeros((L,), jnp.float32)

    def zrow(r, carry):
        for j in range(D // L):
            zbuf[r, pl.ds(j * L, L)] = zv
        return carry

    lax.fori_loop(0, ZROWS, zrow, 0)
    for k in range(ROWS_PER_TILE // ZROWS):
        pltpu.sync_copy(zbuf,
                        acc_sh.at[pl.ds(s * ROWS_PER_TILE + k * ZROWS, ZROWS)])
    plsc.subcore_barrier()

    base = (c * NS + s) * EDGES_PER_TILE

    def chunk_body(i, carry):
        e0 = base + i * CHUNK
        pltpu.sync_copy(src_hbm.at[pl.ds(e0, CHUNK)], src_v)
        pltpu.sync_copy(dst_hbm.at[pl.ds(e0, CHUNK)], dst_v)
        pltpu.sync_copy(val_hbm.at[pl.ds(e0, CHUNK)], val_v)
        pltpu.async_copy(x_hbm.at[src_v], rows_v, sem).wait()

        def scale_grp(g, cc):
            vv16 = val_v[pl.ds(g * L, L)]
            for t in range(L):
                vv = vv16[t]
                e = g * L + t
                for j in range(D // L):
                    sl = pl.ds(j * L, L)
                    rows_v[e, sl] = rows_v[e, sl] * vv
            return cc

        lax.fori_loop(0, CHUNK // L, scale_grp, 0)
        pltpu.sync_copy(rows_v, acc_sh.at[dst_v], add=True)
        return carry

    lax.fori_loop(0, NCHUNK, chunk_body, 0)
    plsc.subcore_barrier()

    w0 = s * WROWS
    pltpu.sync_copy(acc_sh.at[pl.ds(w0, WROWS)],
                    out_hbm.at[c, pl.ds(w0, WROWS)])

    @pl.when(s == 0)
    def _write_tail():
        pltpu.sync_copy(acc_sh.at[pl.ds(NS * WROWS, N - NS * WROWS)],
                        out_hbm.at[c, pl.ds(NS * WROWS, N - NS * WROWS)])


_sc_spmm = functools.partial(
    pl.kernel,
    out_type=jax.ShapeDtypeStruct((NC, N, D), jnp.float32),
    mesh=plsc.VectorSubcoreMesh(core_axis_name="c", subcore_axis_name="s"),
    scratch_types=[
        pltpu.VMEM((CHUNK,), jnp.int32),
        pltpu.VMEM((CHUNK,), jnp.int32),
        pltpu.VMEM((CHUNK,), jnp.float32),
        pltpu.VMEM((CHUNK, D), jnp.float32),
        pltpu.VMEM((ZROWS, D), jnp.float32),
        pltpu.VMEM_SHARED((N, D), jnp.float32),
        pltpu.SemaphoreType.DMA,
    ],
)(_sc_spmm_body)


BLK = 1000


def _linear_body(p_ref, w_ref, b_ref, o_ref):
    acc = p_ref[0] + p_ref[1]
    o_ref[...] = lax.dot_general(
        acc, w_ref[...], (((1,), (1,)), ((), ())),
        preferred_element_type=jnp.float32) + b_ref[...]


_linear = pl.pallas_call(
    _linear_body,
    grid=(N // BLK,),
    in_specs=[
        pl.BlockSpec((NC, BLK, D), lambda i: (0, i, 0)),
        pl.BlockSpec((D, D), lambda i: (0, 0)),
        pl.BlockSpec((1, D), lambda i: (0, 0)),
    ],
    out_specs=pl.BlockSpec((BLK, D), lambda i: (i, 0)),
    out_shape=jax.ShapeDtypeStruct((N, D), jnp.float32),
)


def kernel(A_indices, A_values, X, W, b):
    dst = A_indices[0].astype(jnp.int32)
    src = A_indices[1].astype(jnp.int32)
    partials = _sc_spmm(dst, src, A_values, X)
    return _linear(partials, W, b.reshape(1, D))


# speedup vs baseline: 10.0408x; 2.2087x over previous
import functools

import jax
import jax.numpy as jnp
from jax import lax
from jax.experimental import pallas as pl
from jax.experimental.pallas import tpu as pltpu
from jax.experimental.pallas import tpu_sc as plsc

N = 10000
E = 320000
D = 128

L = 16
NC = 2
NS = 16

CHUNK = 80
EDGES_PER_TILE = E // (NC * NS)
NCHUNK = EDGES_PER_TILE // CHUNK
BLKCH = 25
NBLK = NCHUNK // BLKCH
ROWS_PER_TILE = N // NS
WROWS = 624


def _sc_spmm_body(dst_hbm, src_hbm, val_hbm, x_hbm, out_hbm,
                  src2d, dst2d, val2d, rows0, rows1, acc_sh, gsem0, gsem1):
    c = lax.axis_index("c")
    s = lax.axis_index("s")

    tid = c * NS + s

    zv = jnp.zeros((L,), jnp.float32)

    def zrow(r, carry):
        for j in range(D // L):
            rows0[r, pl.ds(j * L, L)] = zv
        return carry

    lax.fori_loop(0, CHUNK, zrow, 0)
    for k in range(ROWS_PER_TILE // CHUNK):
        pltpu.sync_copy(rows0,
                        acc_sh.at[pl.ds(s * ROWS_PER_TILE + k * CHUNK, CHUNK)])
    ztail = ROWS_PER_TILE % CHUNK
    if ztail:
        pltpu.sync_copy(
            rows0.at[pl.ds(0, ztail)],
            acc_sh.at[pl.ds(s * ROWS_PER_TILE + ROWS_PER_TILE - ztail, ztail)])
    plsc.subcore_barrier()

    rows = (rows0, rows1)
    gsems = (gsem0, gsem1)

    def start_gather(idx, b):
        pltpu.async_copy(x_hbm.at[src2d.at[idx]], rows[b], gsems[b])

    def wait_gather(b):
        pltpu.make_async_copy(x_hbm.at[src2d.at[0]], rows[b], gsems[b]).wait()

    def process(i, b, prefetch):
        wait_gather(b)

        def scale_grp(g, cc):
            vv16 = val2d[i, pl.ds(g * L, L)]
            for t in range(L):
                vv = vv16[t]
                e = g * L + t
                for j in range(D // L):
                    sl = pl.ds(j * L, L)
                    rows[b][e, sl] = rows[b][e, sl] * vv
            return cc

        lax.fori_loop(0, CHUNK // L, scale_grp, 0)
        pltpu.sync_copy(rows[b], acc_sh.at[dst2d.at[i]], add=True)
        if prefetch:
            @pl.when(i + 2 < BLKCH)
            def _():
                start_gather(i + 2, b)

    for kb in range(NBLK):
        pltpu.sync_copy(src_hbm.at[tid, kb], src2d)
        pltpu.sync_copy(dst_hbm.at[tid, kb], dst2d)
        pltpu.sync_copy(val_hbm.at[tid, kb], val2d)
        start_gather(0, 0)
        start_gather(1, 1)

        def outer(io, carry):
            for b in range(2):
                process(io * 2 + b, b, True)
            return carry

        lax.fori_loop(0, BLKCH // 2, outer, 0)
        process(BLKCH - 1, 0, False)
    plsc.subcore_barrier()

    w0 = s * WROWS
    pltpu.sync_copy(acc_sh.at[pl.ds(w0, WROWS)],
                    out_hbm.at[c, pl.ds(w0, WROWS)])

    @pl.when(s == 0)
    def _write_tail():
        pltpu.sync_copy(acc_sh.at[pl.ds(NS * WROWS, N - NS * WROWS)],
                        out_hbm.at[c, pl.ds(NS * WROWS, N - NS * WROWS)])


_sc_spmm = functools.partial(
    pl.kernel,
    out_type=jax.ShapeDtypeStruct((NC, N, D), jnp.float32),
    mesh=plsc.VectorSubcoreMesh(core_axis_name="c", subcore_axis_name="s"),
    scratch_types=[
        pltpu.VMEM((BLKCH, CHUNK), jnp.int32),
        pltpu.VMEM((BLKCH, CHUNK), jnp.int32),
        pltpu.VMEM((BLKCH, CHUNK), jnp.float32),
        pltpu.VMEM((CHUNK, D), jnp.float32),
        pltpu.VMEM((CHUNK, D), jnp.float32),
        pltpu.VMEM_SHARED((N, D), jnp.float32),
        pltpu.SemaphoreType.DMA,
        pltpu.SemaphoreType.DMA,
    ],
)(_sc_spmm_body)


BLK = 1000


def _linear_body(p_ref, w_ref, b_ref, o_ref):
    acc = p_ref[0] + p_ref[1]
    o_ref[...] = lax.dot_general(
        acc, w_ref[...], (((1,), (1,)), ((), ())),
        preferred_element_type=jnp.float32) + b_ref[...]


_linear = pl.pallas_call(
    _linear_body,
    grid=(N // BLK,),
    in_specs=[
        pl.BlockSpec((NC, BLK, D), lambda i: (0, i, 0)),
        pl.BlockSpec((D, D), lambda i: (0, 0)),
        pl.BlockSpec((1, D), lambda i: (0, 0)),
    ],
    out_specs=pl.BlockSpec((BLK, D), lambda i: (i, 0)),
    out_shape=jax.ShapeDtypeStruct((N, D), jnp.float32),
)


def kernel(A_indices, A_values, X, W, b):
    shp = (NC * NS, NBLK, BLKCH, CHUNK)
    dst = A_indices[0].astype(jnp.int32).reshape(shp)
    src = A_indices[1].astype(jnp.int32).reshape(shp)
    vals = A_values.reshape(shp)
    partials = _sc_spmm(dst, src, vals, X)
    return _linear(partials, W, b.reshape(1, D))


# speedup vs baseline: 12.6276x; 1.2576x over previous
import functools

import jax
import jax.numpy as jnp
from jax import lax
from jax.experimental import pallas as pl
from jax.experimental.pallas import tpu as pltpu
from jax.experimental.pallas import tpu_sc as plsc

N = 10000
E = 320000
D = 128

L = 16
NC = 2
NS = 16

CHUNK = 80
EDGES_PER_TILE = E // (NC * NS)
NCHUNK = EDGES_PER_TILE // CHUNK
NBUF = 3
NDST = 4
ROWS_PER_TILE = N // NS
WROWS = 624


def _sc_spmm_body(dst_hbm, src_hbm, val_hbm, x_hbm, out_hbm,
                  srcb, dstb, valb, rows0, rows1, rows2, acc_sh,
                  gsem0, gsem1, gsem2, ssem0, ssem1, ssem2,
                  isem0, isem1, isem2):
    c = lax.axis_index("c")
    s = lax.axis_index("s")

    tid = c * NS + s

    zv = jnp.zeros((L,), jnp.float32)

    def zrow(r, carry):
        for j in range(D // L):
            rows0[r, pl.ds(j * L, L)] = zv
        return carry

    lax.fori_loop(0, CHUNK, zrow, 0)
    for k in range(ROWS_PER_TILE // CHUNK):
        pltpu.sync_copy(rows0,
                        acc_sh.at[pl.ds(s * ROWS_PER_TILE + k * CHUNK, CHUNK)])
    ztail = ROWS_PER_TILE % CHUNK
    if ztail:
        pltpu.sync_copy(
            rows0.at[pl.ds(0, ztail)],
            acc_sh.at[pl.ds(s * ROWS_PER_TILE + ROWS_PER_TILE - ztail, ztail)])
    plsc.subcore_barrier()

    rows = (rows0, rows1, rows2)
    gsems = (gsem0, gsem1, gsem2)
    ssems = (ssem0, ssem1, ssem2)
    isems = (isem0, isem1, isem2)

    ebase = tid * EDGES_PER_TILE

    def start_idx(i, b):
        sl = pl.ds(ebase + i * CHUNK, CHUNK)
        pltpu.async_copy(src_hbm.at[sl], srcb.at[b], isems[b])
        pltpu.async_copy(dst_hbm.at[sl], dstb.at[i % NDST], isems[b])
        pltpu.async_copy(val_hbm.at[sl], valb.at[b], isems[b])

    def wait_idx(b):
        sl = pl.ds(0, CHUNK)
        pltpu.make_async_copy(src_hbm.at[sl], srcb.at[b], isems[b]).wait()
        pltpu.make_async_copy(dst_hbm.at[sl], dstb.at[0], isems[b]).wait()
        pltpu.make_async_copy(val_hbm.at[sl], valb.at[b], isems[b]).wait()

    def start_gather(b):
        pltpu.async_copy(x_hbm.at[srcb.at[b]], rows[b], gsems[b])

    def wait_gather(b):
        pltpu.make_async_copy(x_hbm.at[srcb.at[0]], rows[b], gsems[b]).wait()

    def start_scatter(b, k):
        pltpu.async_copy(rows[b], acc_sh.at[dstb.at[k % NDST]], ssems[b],
                         add=True)

    def wait_scatter(b):
        pltpu.make_async_copy(rows[b], acc_sh.at[dstb.at[0]], ssems[b]).wait()

    for b in range(NBUF):
        start_idx(b, b)
    wait_idx(0)
    start_gather(0)
    wait_idx(1)
    start_gather(1)

    def process(k, rb, first):
        wait_gather(rb)

        def scale_grp(g, cc):
            vv16 = valb[rb, pl.ds(g * L, L)]
            for t in range(L):
                vv = vv16[t]
                e = g * L + t
                for j in range(D // L):
                    sl = pl.ds(j * L, L)
                    rows[rb][e, sl] = rows[rb][e, sl] * vv
            return cc

        lax.fori_loop(0, CHUNK // L, scale_grp, 0)
        start_scatter(rb, k)
        pb = (rb + NBUF - 1) % NBUF
        if not first:
            wait_scatter(pb)

        @pl.when(k + NBUF < NCHUNK)
        def _():
            start_idx(k + NBUF, rb)

        @pl.when(k + 2 < NCHUNK)
        def _():
            wait_idx(pb)
            start_gather(pb)

    process(0, 0, True)
    process(1, 1, False)

    def outer(io, carry):
        k = io * NBUF + 2
        for p in range(NBUF):
            process(k + p, (2 + p) % NBUF, False)
        return carry

    lax.fori_loop(0, (NCHUNK - 2) // NBUF, outer, 0)
    wait_scatter((NCHUNK - 1) % NBUF)
    plsc.subcore_barrier()

    w0 = s * WROWS
    pltpu.sync_copy(acc_sh.at[pl.ds(w0, WROWS)],
                    out_hbm.at[c, pl.ds(w0, WROWS)])

    @pl.when(s == 0)
    def _write_tail():
        pltpu.sync_copy(acc_sh.at[pl.ds(NS * WROWS, N - NS * WROWS)],
                        out_hbm.at[c, pl.ds(NS * WROWS, N - NS * WROWS)])


_sc_spmm = functools.partial(
    pl.kernel,
    out_type=jax.ShapeDtypeStruct((NC, N, D), jnp.float32),
    mesh=plsc.VectorSubcoreMesh(core_axis_name="c", subcore_axis_name="s"),
    scratch_types=[
        pltpu.VMEM((NBUF, CHUNK), jnp.int32),
        pltpu.VMEM((NDST, CHUNK), jnp.int32),
        pltpu.VMEM((NBUF, CHUNK), jnp.float32),
        pltpu.VMEM((CHUNK, D), jnp.float32),
        pltpu.VMEM((CHUNK, D), jnp.float32),
        pltpu.VMEM((CHUNK, D), jnp.float32),
        pltpu.VMEM_SHARED((N, D), jnp.float32),
        pltpu.SemaphoreType.DMA,
        pltpu.SemaphoreType.DMA,
        pltpu.SemaphoreType.DMA,
        pltpu.SemaphoreType.DMA,
        pltpu.SemaphoreType.DMA,
        pltpu.SemaphoreType.DMA,
        pltpu.SemaphoreType.DMA,
        pltpu.SemaphoreType.DMA,
        pltpu.SemaphoreType.DMA,
    ],
)(_sc_spmm_body)


BLK = 1000


def _linear_body(p_ref, w_ref, b_ref, o_ref):
    acc = p_ref[0] + p_ref[1]
    o_ref[...] = lax.dot_general(
        acc, w_ref[...], (((1,), (1,)), ((), ())),
        preferred_element_type=jnp.float32) + b_ref[...]


_linear = pl.pallas_call(
    _linear_body,
    grid=(N // BLK,),
    in_specs=[
        pl.BlockSpec((NC, BLK, D), lambda i: (0, i, 0)),
        pl.BlockSpec((D, D), lambda i: (0, 0)),
        pl.BlockSpec((1, D), lambda i: (0, 0)),
    ],
    out_specs=pl.BlockSpec((BLK, D), lambda i: (i, 0)),
    out_shape=jax.ShapeDtypeStruct((N, D), jnp.float32),
)


def kernel(A_indices, A_values, X, W, b):
    dst = A_indices[0].astype(jnp.int32)
    src = A_indices[1].astype(jnp.int32)
    vals = A_values
    partials = _sc_spmm(dst, src, vals, X)
    return _linear(partials, W, b.reshape(1, D))
